# baseline (speedup 1.0000x reference)
import functools

import jax
import jax.numpy as jnp
from jax import lax
from jax.experimental import pallas as pl
from jax.experimental.pallas import tpu as pltpu
from jax.experimental.pallas import tpu_sc as plsc

NC = 2
NS = 16
NW = NC * NS
LANES = 16


def _sc_hist(dst, n_nodes):
    (E,) = dst.shape
    epw = E // NW
    assert E % NW == 0 and epw % LANES == 0

    mesh = plsc.VectorSubcoreMesh(core_axis_name="c", subcore_axis_name="s",
                                  num_cores=NC, num_subcores=NS)

    @functools.partial(
        pl.kernel,
        mesh=mesh,
        out_type=jax.ShapeDtypeStruct((NW * n_nodes,), jnp.float32),
        compiler_params=pltpu.CompilerParams(needs_layout_passes=False),
        scratch_types=[
            pltpu.VMEM((epw,), jnp.int32),
            pltpu.VMEM((n_nodes,), jnp.float32),
        ],
    )
    def hist_kernel(dst_hbm, out_hbm, dst_v, hist_v):
        c = lax.axis_index("c")
        s = lax.axis_index("s")
        wid = c * NS + s
        pltpu.sync_copy(dst_hbm.at[pl.ds(wid * epw, epw)], dst_v)

        zeros = jnp.zeros((LANES,), jnp.float32)

        def zero_body(i, _):
            hist_v[pl.ds(i * LANES, LANES)] = zeros
            return 0

        lax.fori_loop(0, n_nodes // LANES, zero_body, 0)

        ones = jnp.ones((LANES,), jnp.float32)

        def acc_body(i, _):
            idx = dst_v[pl.ds(i * LANES, LANES)]
            plsc.addupdate_scatter(hist_v, [idx], ones)
            return 0

        lax.fori_loop(0, epw // LANES, acc_body, 0)
        pltpu.sync_copy(hist_v, out_hbm.at[pl.ds(wid * n_nodes, n_nodes)])

    return hist_kernel(dst).reshape(NW, n_nodes)


def _sc_scatter(src, dst, h2):
    (E,) = src.shape
    n_nodes, d = h2.shape
    epw = E // NW
    CHUNK = 80
    assert epw % CHUNK == 0
    n_chunks = epw // CHUNK
    rpt = -(-n_nodes // NS // CHUNK) * CHUNK
    n_pad = rpt * NS

    mesh = plsc.VectorSubcoreMesh(core_axis_name="c", subcore_axis_name="s",
                                  num_cores=NC, num_subcores=NS)

    @functools.partial(
        pl.kernel,
        mesh=mesh,
        out_type=jax.ShapeDtypeStruct((NC, n_pad, d), jnp.float32),
        compiler_params=pltpu.CompilerParams(needs_layout_passes=False),
        scratch_types=[
            pltpu.VMEM((epw,), jnp.int32),
            pltpu.VMEM((CHUNK,), jnp.int32),
            pltpu.VMEM((CHUNK,), jnp.int32),
            pltpu.VMEM((CHUNK, d), jnp.float32),
            pltpu.VMEM((CHUNK, d), jnp.float32),
            pltpu.VMEM_SHARED((n_pad, d), jnp.float32),
            pltpu.SemaphoreType.DMA,
            pltpu.SemaphoreType.DMA,
            pltpu.SemaphoreType.DMA,
            pltpu.SemaphoreType.DMA,
            pltpu.SemaphoreType.DMA,
            pltpu.SemaphoreType.DMA,
        ],
    )
    def scatter_kernel(src_hbm, dst_hbm, h2_hbm, out_hbm,
                       src_all, dst_v0, dst_v1, rows0, rows1, acc,
                       sg0, sg1, sd0, sd1, ss0, ss1):
        c = lax.axis_index("c")
        s = lax.axis_index("s")
        wid = c * NS + s
        ebase = wid * epw

        zeros = jnp.zeros((LANES,), jnp.float32)
        vecs_per_row = d // LANES

        def zbuf_body(k, _):
            i = k // vecs_per_row
            j = k % vecs_per_row
            rows0[i, pl.ds(j * LANES, LANES)] = zeros
            return 0

        lax.fori_loop(0, CHUNK * vecs_per_row, zbuf_body, 0)

        row0 = s * rpt
        for t in range(rpt // CHUNK):
            pltpu.sync_copy(rows0, acc.at[pl.ds(row0 + t * CHUNK, CHUNK)])
        plsc.subcore_barrier()

        pltpu.sync_copy(src_hbm.at[pl.ds(ebase, epw)], src_all)

        def gather_start(k, rows, sem):
            pltpu.async_copy(
                h2_hbm.at[src_all.at[pl.ds(k * CHUNK, CHUNK)]], rows, sem)

        def dst_start(k, dv, sem):
            pltpu.async_copy(dst_hbm.at[pl.ds(ebase + k * CHUNK, CHUNK)],
                             dv, sem)

        gather_start(0, rows0, sg0)
        dst_start(0, dst_v0, sd0)

        def process(k, rows, sem_g, dv, sem_d, sem_s,
                    o_rows, o_sg, o_dv, o_sd, o_ss):
            pltpu.make_async_copy(
                h2_hbm.at[src_all.at[pl.ds(k * CHUNK, CHUNK)]],
                rows, sem_g).wait()

            @pl.when(k >= 1)
            def _():
                pltpu.make_async_copy(o_rows, acc.at[o_dv], o_ss).wait()

            @pl.when(k + 1 < n_chunks)
            def _():
                gather_start(k + 1, o_rows, o_sg)
                dst_start(k + 1, o_dv, o_sd)

            pltpu.make_async_copy(
                dst_hbm.at[pl.ds(ebase + k * CHUNK, CHUNK)], dv, sem_d).wait()
            pltpu.async_copy(rows, acc.at[dv], sem_s, add=True)

        def edge_body(k, _):
            @pl.when(k % 2 == 0)
            def _():
                process(k, rows0, sg0, dst_v0, sd0, ss0,
                        rows1, sg1, dst_v1, sd1, ss1)

            @pl.when(k % 2 == 1)
            def _():
                process(k, rows1, sg1, dst_v1, sd1, ss1,
                        rows0, sg0, dst_v0, sd0, ss0)

            return 0

        lax.fori_loop(0, n_chunks, edge_body, 0)
        if n_chunks % 2 == 1:
            pltpu.make_async_copy(rows0, acc.at[dst_v0], ss0).wait()
        else:
            pltpu.make_async_copy(rows1, acc.at[dst_v1], ss1).wait()
        plsc.subcore_barrier()

        pltpu.sync_copy(acc.at[pl.ds(row0, rpt)], out_hbm.at[c, pl.ds(row0, rpt)])

    return scatter_kernel(src, dst, h2)


def _tc_dense(x, W_in, b_in, W_h, b_h, W_g, hist):
    n, d_in = x.shape
    d_h = W_in.shape[1]

    def body(x_ref, wi_ref, bi_ref, wh_ref, bh_ref, wg_ref, hist_ref,
             feat_ref, h2_ref):
        f1 = jnp.maximum(
            jnp.dot(x_ref[...], wi_ref[...],
                    preferred_element_type=jnp.float32) + bi_ref[...], 0.0)
        f2 = jnp.maximum(
            jnp.dot(f1, wh_ref[...],
                    preferred_element_type=jnp.float32) + bh_ref[...], 0.0)
        h = jnp.dot(f2, wg_ref[...], preferred_element_type=jnp.float32)
        deg = jnp.sum(hist_ref[...], axis=0) + 1.0
        dinv = lax.rsqrt(deg)
        feat_ref[...] = f2
        h2_ref[...] = h * dinv[:, None]

    feat, h2 = pl.pallas_call(
        body,
        out_shape=[
            jax.ShapeDtypeStruct((n, d_h), jnp.float32),
            jax.ShapeDtypeStruct((n, d_h), jnp.float32),
        ],
    )(x, W_in, b_in.reshape(1, -1), W_h, b_h.reshape(1, -1), W_g, hist)
    return feat, h2


def _tc_finish(p, h2, hist, b_g):
    n, d = h2.shape

    def body(p_ref, h2_ref, hist_ref, bg_ref, out_ref):
        deg = jnp.sum(hist_ref[...], axis=0) + 1.0
        dinv = lax.rsqrt(deg)
        tot = p_ref[0, :n, :] + p_ref[1, :n, :] + h2_ref[...]
        out_ref[...] = tot * dinv[:, None] + bg_ref[...]

    return pl.pallas_call(
        body,
        out_shape=jax.ShapeDtypeStruct((n, d), jnp.float32),
    )(p, h2, hist, b_g.reshape(1, -1))


def kernel(x, edge_index, W_in, b_in, W_h, b_h, W_g, b_g):
    n = x.shape[0]
    src = edge_index[0]
    dst = edge_index[1]
    hist = _sc_hist(dst, n)
    feat, h2 = _tc_dense(x, W_in, b_in, W_h, b_h, W_g, hist)
    p = _sc_scatter(src, dst, h2)
    out_feat = _tc_finish(p, h2, hist, b_g)
    return (feat, out_feat)


# speedup vs baseline: 1.3445x; 1.3445x over previous
import functools

import jax
import jax.numpy as jnp
from jax import lax
from jax.experimental import pallas as pl
from jax.experimental.pallas import tpu as pltpu
from jax.experimental.pallas import tpu_sc as plsc

NC = 2
NS = 16
NW = NC * NS
LANES = 16


def _sc_hist(dst, n_nodes):
    (E,) = dst.shape
    epw = E // NW
    assert E % NW == 0 and epw % LANES == 0

    mesh = plsc.VectorSubcoreMesh(core_axis_name="c", subcore_axis_name="s",
                                  num_cores=NC, num_subcores=NS)

    @functools.partial(
        pl.kernel,
        mesh=mesh,
        out_type=jax.ShapeDtypeStruct((NW * n_nodes,), jnp.float32),
        compiler_params=pltpu.CompilerParams(needs_layout_passes=False),
        scratch_types=[
            pltpu.VMEM((epw,), jnp.int32),
            pltpu.VMEM((n_nodes,), jnp.float32),
        ],
    )
    def hist_kernel(dst_hbm, out_hbm, dst_v, hist_v):
        c = lax.axis_index("c")
        s = lax.axis_index("s")
        wid = c * NS + s
        pltpu.sync_copy(dst_hbm.at[pl.ds(wid * epw, epw)], dst_v)

        zeros = jnp.zeros((LANES,), jnp.float32)

        def zero_body(i, _):
            hist_v[pl.ds(i * LANES, LANES)] = zeros
            return 0

        lax.fori_loop(0, n_nodes // LANES, zero_body, 0)

        ones = jnp.ones((LANES,), jnp.float32)

        def acc_body(i, _):
            idx = dst_v[pl.ds(i * LANES, LANES)]
            plsc.addupdate_scatter(hist_v, [idx], ones)
            return 0

        lax.fori_loop(0, epw // LANES, acc_body, 0)
        pltpu.sync_copy(hist_v, out_hbm.at[pl.ds(wid * n_nodes, n_nodes)])

    return hist_kernel(dst).reshape(NW, n_nodes)


def _sc_scatter(src, dst, h2):
    (E,) = src.shape
    n_nodes, d = h2.shape
    epw = E // NW
    CHUNK = 80
    assert epw % CHUNK == 0
    n_chunks = epw // CHUNK
    rpt = -(-n_nodes // NS // CHUNK) * CHUNK
    n_pad = rpt * NS

    mesh = plsc.VectorSubcoreMesh(core_axis_name="c", subcore_axis_name="s",
                                  num_cores=NC, num_subcores=NS)

    @functools.partial(
        pl.kernel,
        mesh=mesh,
        out_type=jax.ShapeDtypeStruct((NC, n_pad, d), jnp.float32),
        compiler_params=pltpu.CompilerParams(needs_layout_passes=False),
        scratch_types=[
            pltpu.VMEM((epw,), jnp.int32),
            pltpu.VMEM((CHUNK,), jnp.int32),
            pltpu.VMEM((CHUNK,), jnp.int32),
            pltpu.VMEM((CHUNK,), jnp.int32),
            pltpu.VMEM((CHUNK, d), jnp.float32),
            pltpu.VMEM((CHUNK, d), jnp.float32),
            pltpu.VMEM((CHUNK, d), jnp.float32),
            pltpu.VMEM_SHARED((n_pad, d), jnp.float32),
        ] + [pltpu.SemaphoreType.DMA] * 9,
    )
    def scatter_kernel(src_hbm, dst_hbm, h2_hbm, out_hbm,
                       src_all, dv0, dv1, dv2, rows0, rows1, rows2, acc,
                       sg0, sg1, sg2, sd0, sd1, sd2, ss0, ss1, ss2):
        c = lax.axis_index("c")
        s = lax.axis_index("s")
        wid = c * NS + s
        ebase = wid * epw
        slots = ((rows0, dv0, sg0, sd0, ss0),
                 (rows1, dv1, sg1, sd1, ss1),
                 (rows2, dv2, sg2, sd2, ss2))

        zeros = jnp.zeros((LANES,), jnp.float32)
        vecs_per_row = d // LANES

        def zbuf_body(k, _):
            i = k // vecs_per_row
            j = k % vecs_per_row
            rows0[i, pl.ds(j * LANES, LANES)] = zeros
            return 0

        lax.fori_loop(0, CHUNK * vecs_per_row, zbuf_body, 0)

        row0 = s * rpt
        for t in range(rpt // CHUNK):
            pltpu.sync_copy(rows0, acc.at[pl.ds(row0 + t * CHUNK, CHUNK)])
        plsc.subcore_barrier()

        pltpu.sync_copy(src_hbm.at[pl.ds(ebase, epw)], src_all)

        def gather_start(k, slot):
            rows, dv, sg, sd, _ = slot
            pltpu.async_copy(
                h2_hbm.at[src_all.at[pl.ds(k * CHUNK, CHUNK)]], rows, sg)
            pltpu.async_copy(dst_hbm.at[pl.ds(ebase + k * CHUNK, CHUNK)],
                             dv, sd)

        gather_start(0, slots[0])
        gather_start(1, slots[1])

        def process(k, slot, nxt):
            rows, dv, sg, sd, ss = slot
            n_rows, n_dv, _, _, n_ss = nxt

            @pl.when(k >= 2)
            def _():
                pltpu.make_async_copy(n_rows, acc.at[n_dv], n_ss).wait()

            @pl.when((k >= 1) & (k + 1 < n_chunks))
            def _():
                gather_start(k + 1, nxt)

            pltpu.make_async_copy(
                h2_hbm.at[src_all.at[pl.ds(k * CHUNK, CHUNK)]],
                rows, sg).wait()
            pltpu.make_async_copy(
                dst_hbm.at[pl.ds(ebase + k * CHUNK, CHUNK)], dv, sd).wait()
            pltpu.async_copy(rows, acc.at[dv], ss, add=True)

        def edge_body(k, _):
            for p in range(3):
                @pl.when(k % 3 == p)
                def _(p=p):
                    process(k, slots[p], slots[(p + 1) % 3])
            return 0

        lax.fori_loop(0, n_chunks, edge_body, 0)
        for k in (n_chunks - 2, n_chunks - 1):
            rows, dv, _, _, ss = slots[k % 3]
            pltpu.make_async_copy(rows, acc.at[dv], ss).wait()
        plsc.subcore_barrier()

        pltpu.sync_copy(acc.at[pl.ds(row0, rpt)], out_hbm.at[c, pl.ds(row0, rpt)])

    return scatter_kernel(src, dst, h2)


def _tc_dense(x, W_in, b_in, W_h, b_h, W_g, hist):
    n, d_in = x.shape
    d_h = W_in.shape[1]

    def body(x_ref, wi_ref, bi_ref, wh_ref, bh_ref, wg_ref, hist_ref,
             feat_ref, h2_ref):
        f1 = jnp.maximum(
            jnp.dot(x_ref[...], wi_ref[...],
                    preferred_element_type=jnp.float32) + bi_ref[...], 0.0)
        f2 = jnp.maximum(
            jnp.dot(f1, wh_ref[...],
                    preferred_element_type=jnp.float32) + bh_ref[...], 0.0)
        h = jnp.dot(f2, wg_ref[...], preferred_element_type=jnp.float32)
        deg = jnp.sum(hist_ref[...], axis=0) + 1.0
        dinv = lax.rsqrt(deg)
        feat_ref[...] = f2
        h2_ref[...] = h * dinv[:, None]

    feat, h2 = pl.pallas_call(
        body,
        out_shape=[
            jax.ShapeDtypeStruct((n, d_h), jnp.float32),
            jax.ShapeDtypeStruct((n, d_h), jnp.float32),
        ],
    )(x, W_in, b_in.reshape(1, -1), W_h, b_h.reshape(1, -1), W_g, hist)
    return feat, h2


def _tc_finish(p, h2, hist, b_g):
    n, d = h2.shape

    def body(p_ref, h2_ref, hist_ref, bg_ref, out_ref):
        deg = jnp.sum(hist_ref[...], axis=0) + 1.0
        dinv = lax.rsqrt(deg)
        tot = p_ref[0, :n, :] + p_ref[1, :n, :] + h2_ref[...]
        out_ref[...] = tot * dinv[:, None] + bg_ref[...]

    return pl.pallas_call(
        body,
        out_shape=jax.ShapeDtypeStruct((n, d), jnp.float32),
    )(p, h2, hist, b_g.reshape(1, -1))


def kernel(x, edge_index, W_in, b_in, W_h, b_h, W_g, b_g):
    n = x.shape[0]
    src = edge_index[0]
    dst = edge_index[1]
    hist = _sc_hist(dst, n)
    feat, h2 = _tc_dense(x, W_in, b_in, W_h, b_h, W_g, hist)
    p = _sc_scatter(src, dst, h2)
    out_feat = _tc_finish(p, h2, hist, b_g)
    return (feat, out_feat)


# speedup vs baseline: 1.3658x; 1.0158x over previous
import functools

import jax
import jax.numpy as jnp
from jax import lax
from jax.experimental import pallas as pl
from jax.experimental.pallas import tpu as pltpu
from jax.experimental.pallas import tpu_sc as plsc

NC = 2
NS = 16
NW = NC * NS
LANES = 16


def _sc_hist(dst, n_nodes):
    (E,) = dst.shape
    epw = E // NW
    assert E % NW == 0 and epw % LANES == 0

    mesh = plsc.VectorSubcoreMesh(core_axis_name="c", subcore_axis_name="s",
                                  num_cores=NC, num_subcores=NS)

    @functools.partial(
        pl.kernel,
        mesh=mesh,
        out_type=jax.ShapeDtypeStruct((NW * n_nodes,), jnp.float32),
        compiler_params=pltpu.CompilerParams(needs_layout_passes=False),
        scratch_types=[
            pltpu.VMEM((epw,), jnp.int32),
            pltpu.VMEM((n_nodes,), jnp.float32),
        ],
    )
    def hist_kernel(dst_hbm, out_hbm, dst_v, hist_v):
        c = lax.axis_index("c")
        s = lax.axis_index("s")
        wid = c * NS + s
        pltpu.sync_copy(dst_hbm.at[pl.ds(wid * epw, epw)], dst_v)

        zeros = jnp.zeros((LANES,), jnp.float32)

        def zero_body(i, _):
            hist_v[pl.ds(i * LANES, LANES)] = zeros
            return 0

        lax.fori_loop(0, n_nodes // LANES, zero_body, 0)

        ones = jnp.ones((LANES,), jnp.float32)

        def acc_body(i, _):
            idx = dst_v[pl.ds(i * LANES, LANES)]
            plsc.addupdate_scatter(hist_v, [idx], ones)
            return 0

        lax.fori_loop(0, epw // LANES, acc_body, 0)
        pltpu.sync_copy(hist_v, out_hbm.at[pl.ds(wid * n_nodes, n_nodes)])

    return hist_kernel(dst).reshape(NW, n_nodes)


def _sc_scatter(src, dst, h2, n_nodes):
    (E,) = src.shape
    d = h2.shape[1]
    epw = E // NW
    CHUNK = 80
    assert epw % CHUNK == 0
    n_chunks = epw // CHUNK
    rpt = -(-n_nodes // NS // CHUNK) * CHUNK
    n_pad = rpt * NS

    mesh = plsc.VectorSubcoreMesh(core_axis_name="c", subcore_axis_name="s",
                                  num_cores=NC, num_subcores=NS)

    @functools.partial(
        pl.kernel,
        mesh=mesh,
        out_type=jax.ShapeDtypeStruct((NC, n_pad, d), jnp.float32),
        compiler_params=pltpu.CompilerParams(needs_layout_passes=False),
        scratch_types=[
            pltpu.VMEM((epw,), jnp.int32),
            pltpu.VMEM((CHUNK,), jnp.int32),
            pltpu.VMEM((CHUNK,), jnp.int32),
            pltpu.VMEM((CHUNK,), jnp.int32),
            pltpu.VMEM((CHUNK, d), jnp.float32),
            pltpu.VMEM((CHUNK, d), jnp.float32),
            pltpu.VMEM((CHUNK, d), jnp.float32),
            pltpu.VMEM_SHARED((n_pad, d), jnp.float32),
        ] + [pltpu.SemaphoreType.DMA] * 9,
    )
    def scatter_kernel(src_hbm, dst_hbm, h2_hbm, out_hbm,
                       src_all, dv0, dv1, dv2,
                       rows0, rows1, rows2, acc,
                       sg0, sg1, sg2, sd0, sd1, sd2, ss0, ss1, ss2):
        c = lax.axis_index("c")
        s = lax.axis_index("s")
        wid = c * NS + s
        ebase = wid * epw
        slots = ((rows0, dv0, sg0, sd0, ss0),
                 (rows1, dv1, sg1, sd1, ss1),
                 (rows2, dv2, sg2, sd2, ss2))

        zeros = jnp.zeros((LANES,), jnp.float32)
        vecs_per_row = d // LANES

        def zbuf_body(k, _):
            i = k // vecs_per_row
            j = k % vecs_per_row
            rows0[i, pl.ds(j * LANES, LANES)] = zeros
            return 0

        lax.fori_loop(0, CHUNK * vecs_per_row, zbuf_body, 0)

        row0 = s * rpt
        for t in range(rpt // CHUNK):
            pltpu.sync_copy(rows0, acc.at[pl.ds(row0 + t * CHUNK, CHUNK)])
        plsc.subcore_barrier()

        pltpu.sync_copy(src_hbm.at[pl.ds(ebase, epw)], src_all)

        def gather_start(k, slot):
            rows, dv, sg, sd, _ = slot
            pltpu.async_copy(
                h2_hbm.at[src_all.at[pl.ds(k * CHUNK, CHUNK)]], rows, sg)
            pltpu.async_copy(dst_hbm.at[pl.ds(ebase + k * CHUNK, CHUNK)],
                             dv, sd)

        gather_start(0, slots[0])
        gather_start(1, slots[1])

        def process(k, slot, nxt):
            rows, dv, sg, sd, ss = slot
            n_rows, n_dv, _, _, n_ss = nxt

            @pl.when(k >= 2)
            def _():
                pltpu.make_async_copy(n_rows, acc.at[n_dv], n_ss).wait()

            @pl.when((k >= 1) & (k + 1 < n_chunks))
            def _():
                gather_start(k + 1, nxt)

            pltpu.make_async_copy(
                h2_hbm.at[src_all.at[pl.ds(k * CHUNK, CHUNK)]],
                rows, sg).wait()
            pltpu.make_async_copy(
                dst_hbm.at[pl.ds(ebase + k * CHUNK, CHUNK)], dv, sd).wait()
            pltpu.async_copy(rows, acc.at[dv], ss, add=True)

        def edge_body(k, _):
            for p in range(3):
                @pl.when(k % 3 == p)
                def _(p=p):
                    process(k, slots[p], slots[(p + 1) % 3])
            return 0

        lax.fori_loop(0, n_chunks, edge_body, 0)
        for k in (n_chunks - 2, n_chunks - 1):
            rows, dv, _, _, ss = slots[k % 3]
            pltpu.make_async_copy(rows, acc.at[dv], ss).wait()
        plsc.subcore_barrier()

        pltpu.sync_copy(acc.at[pl.ds(row0, rpt)], out_hbm.at[c, pl.ds(row0, rpt)])

    return scatter_kernel(src, dst, h2)


def _tc_dense(x, W_in, b_in, W_h, b_h, W_g):
    n, d_in = x.shape
    d_h = W_in.shape[1]

    def body(x_ref, wi_ref, bi_ref, wh_ref, bh_ref, wg_ref,
             feat_ref, h_ref):
        f1 = jnp.maximum(
            jnp.dot(x_ref[...], wi_ref[...],
                    preferred_element_type=jnp.float32) + bi_ref[...], 0.0)
        f2 = jnp.maximum(
            jnp.dot(f1, wh_ref[...],
                    preferred_element_type=jnp.float32) + bh_ref[...], 0.0)
        feat_ref[...] = f2
        h_ref[...] = jnp.dot(f2, wg_ref[...],
                             preferred_element_type=jnp.float32)

    feat, h = pl.pallas_call(
        body,
        out_shape=[
            jax.ShapeDtypeStruct((n, d_h), jnp.float32),
            jax.ShapeDtypeStruct((n, d_h), jnp.float32),
        ],
    )(x, W_in, b_in.reshape(1, -1), W_h, b_h.reshape(1, -1), W_g)
    return feat, h


def _tc_scale(h, hist):
    n, d = h.shape

    def body(h_ref, hist_ref, h2_ref):
        deg = jnp.sum(hist_ref[...], axis=0) + 1.0
        dinv = lax.rsqrt(deg)
        h2_ref[...] = h_ref[...] * dinv[:, None]

    return pl.pallas_call(
        body,
        out_shape=jax.ShapeDtypeStruct((n, d), jnp.float32),
    )(h, hist)


def _tc_finish(p, h2, hist, b_g):
    n, d = h2.shape

    def body(p_ref, h2_ref, hist_ref, bg_ref, out_ref):
        deg = jnp.sum(hist_ref[...], axis=0) + 1.0
        dinv = lax.rsqrt(deg)
        tot = p_ref[0, :n, :] + p_ref[1, :n, :] + h2_ref[...]
        out_ref[...] = tot * dinv[:, None] + bg_ref[...]

    return pl.pallas_call(
        body,
        out_shape=jax.ShapeDtypeStruct((n, d), jnp.float32),
    )(p, h2, hist, b_g.reshape(1, -1))


def kernel(x, edge_index, W_in, b_in, W_h, b_h, W_g, b_g):
    n = x.shape[0]
    src = edge_index[0]
    dst = edge_index[1]
    hist = _sc_hist(dst, n)
    feat, h = _tc_dense(x, W_in, b_in, W_h, b_h, W_g)
    h2 = _tc_scale(h, hist)
    p = _sc_scatter(src, dst, h2, n)
    out_feat = _tc_finish(p, h2, hist, b_g)
    return (feat, out_feat)
